# initial kernel scaffold (unmeasured)
import jax
import jax.numpy as jnp
from jax import lax
from jax.experimental import pallas as pl
from jax.experimental.pallas import tpu as pltpu


def kernel(
    t,
):
    def body(*refs):
        pass

    out_shape = jax.ShapeDtypeStruct(..., jnp.float32)
    return pl.pallas_call(body, out_shape=out_shape)(...)


# baseline (device time: 704674 ns/iter reference)
import jax
import jax.numpy as jnp
from jax import lax
from jax.experimental import pallas as pl
from jax.experimental.pallas import tpu as pltpu

N_DEV = 4
M = 8192
N = 2048
MB = M // N_DEV
C = 512
N_SUB = MB // C


def kernel(t):
    f32 = jnp.float32
    bf16 = jnp.bfloat16

    def body(t_ref, out_ref, ld_ref, rs_send, rs_recv, ag_bf, ag_recv,
             acc_ref, rs_ssem, rs_rsem, ag_ssem, ag_rsem, ld_sem, st_sem):
        pos = lax.axis_index("i")
        left = (pos - 1) % N_DEV
        right = (pos + 1) % N_DEV

        barrier_sem = pltpu.get_barrier_semaphore()
        for nbr in (left, right):
            pl.semaphore_signal(
                barrier_sem, inc=1,
                device_id=(nbr,), device_id_type=pl.DeviceIdType.MESH,
            )
        pl.semaphore_wait(barrier_sem, 2)

        def load_block(b, c):
            cp = pltpu.make_async_copy(
                t_ref.at[pl.ds(b * MB + c * C, C)], ld_ref, ld_sem)
            cp.start()
            cp.wait()

        def store_block(b, c):
            cp = pltpu.make_async_copy(
                acc_ref, out_ref.at[pl.ds(b * MB + c * C, C)], st_sem)
            cp.start()
            cp.wait()

        for c in range(N_SUB):
            load_block(pos, c)
            rs_send[0, :, :] = ld_ref[:, :].astype(bf16)
            for h in range(N_DEV - 1):
                g = 3 * c + h
                rdma = pltpu.make_async_remote_copy(
                    src_ref=rs_send.at[h % 2],
                    dst_ref=rs_recv.at[g % 4],
                    send_sem=rs_ssem.at[h % 2],
                    recv_sem=rs_rsem.at[g % 4],
                    device_id=(right,),
                    device_id_type=pl.DeviceIdType.MESH,
                )
                rdma.start()
                rdma.wait()
                bb = (pos - 1 - h) % N_DEV
                load_block(bb, c)
                if h < N_DEV - 2:
                    rs_send[(h + 1) % 2, :, :] = (
                        rs_recv[g % 4, :, :].astype(f32) + ld_ref[:, :]
                    ).astype(bf16)
                else:
                    s = rs_recv[g % 4, :, :].astype(f32) + ld_ref[:, :]
                    r = jnp.maximum(s, 0.0)
                    acc_ref[:, :] = jnp.tanh(s) * s * s + r * r * r

            mine = (pos + 1) % N_DEV
            store_block(mine, c)
            ag_bf[0, :, :] = acc_ref[:, :].astype(bf16)

            for h in range(N_DEV - 1):
                g = 3 * c + h
                src = ag_bf.at[0] if h == 0 else ag_recv.at[(g - 1) % 4]
                rdma = pltpu.make_async_remote_copy(
                    src_ref=src,
                    dst_ref=ag_recv.at[g % 4],
                    send_sem=ag_ssem.at[h % 2],
                    recv_sem=ag_rsem.at[g % 4],
                    device_id=(right,),
                    device_id_type=pl.DeviceIdType.MESH,
                )
                rdma.start()
                rdma.wait()
                bb = (pos - h) % N_DEV
                acc_ref[:, :] = ag_recv[g % 4, :, :].astype(f32)
                store_block(bb, c)

    return pl.pallas_call(
        body,
        out_shape=jax.ShapeDtypeStruct((M, N), f32),
        in_specs=[pl.BlockSpec(memory_space=pl.ANY)],
        out_specs=pl.BlockSpec(memory_space=pl.ANY),
        scratch_shapes=[
            pltpu.VMEM((C, N), f32),
            pltpu.VMEM((2, C, N), bf16),
            pltpu.VMEM((4, C, N), bf16),
            pltpu.VMEM((1, C, N), bf16),
            pltpu.VMEM((4, C, N), bf16),
            pltpu.VMEM((C, N), f32),
            pltpu.SemaphoreType.DMA((2,)),
            pltpu.SemaphoreType.DMA((4,)),
            pltpu.SemaphoreType.DMA((2,)),
            pltpu.SemaphoreType.DMA((4,)),
            pltpu.SemaphoreType.DMA,
            pltpu.SemaphoreType.DMA,
        ],
        compiler_params=pltpu.CompilerParams(collective_id=0),
    )(t)


# device time: 396007 ns/iter; 1.7794x vs baseline; 1.7794x over previous
import jax
import jax.numpy as jnp
from jax import lax
from jax.experimental import pallas as pl
from jax.experimental.pallas import tpu as pltpu

N_DEV = 4
M = 8192
N = 2048
HN = N // 2
MB = M // N_DEV
C = 512
N_SUB = MB // C

f32 = jnp.float32
bf16 = jnp.bfloat16


def kernel(t):
    def body(t_ref, out_ref, ld, rs_send, rs_recv, ag_bf, ag_recv, acc,
             rs_ssem, rs_rsem, ag_ssem, ag_rsem, ld_sem, st_sem):
        pos = lax.axis_index("i")
        left = (pos - 1) % N_DEV
        right = (pos + 1) % N_DEV

        barrier_sem = pltpu.get_barrier_semaphore()
        for nbr in (left, right):
            pl.semaphore_signal(
                barrier_sem, inc=1,
                device_id=(nbr,), device_id_type=pl.DeviceIdType.MESH,
            )
        pl.semaphore_wait(barrier_sem, 2)

        DIRS = (
            dict(d=0, col0=0, sgn=1, to=right),
            dict(d=1, col0=HN, sgn=-1, to=left),
        )

        def start_load(dr, b):
            cp = pltpu.make_async_copy(
                t_ref.at[pl.ds(b * MB + _c * C, C), pl.ds(dr["col0"], HN)],
                ld.at[dr["d"]], ld_sem.at[dr["d"]])
            cp.start()
            return cp

        acc_uses = [0, 0]
        pending_st = {}

        def acc_write(dr, value):
            d = dr["d"]
            slot = acc_uses[d] % 2
            acc_uses[d] += 1
            if (d, slot) in pending_st:
                pending_st.pop((d, slot)).wait()
            acc[d, slot, :, :] = value
            return slot

        def start_store(dr, slot, b):
            d = dr["d"]
            cp = pltpu.make_async_copy(
                acc.at[d, slot],
                out_ref.at[pl.ds(b * MB + _c * C, C), pl.ds(dr["col0"], HN)],
                st_sem.at[d, slot])
            cp.start()
            pending_st[(d, slot)] = cp

        def rdma(dr, src, dst_slot, sems_s, slot_s, sems_r):
            return pltpu.make_async_remote_copy(
                src_ref=src, dst_ref=dst_slot,
                send_sem=sems_s.at[dr["d"], slot_s],
                recv_sem=sems_r,
                device_id=(dr["to"],), device_id_type=pl.DeviceIdType.MESH,
            )

        for _c in range(N_SUB):
            for dr in DIRS:
                cp = start_load(dr, pos)
                cp.wait()
                rs_send[dr["d"], 0, :, :] = ld[dr["d"], :, :].astype(bf16)
            for h in range(N_DEV - 1):
                g = 3 * _c + h
                rdmas, loads = [], []
                for dr in DIRS:
                    r = rdma(dr, rs_send.at[dr["d"], h % 2],
                             rs_recv.at[dr["d"], g % 4],
                             rs_ssem, h % 2, rs_rsem.at[dr["d"], g % 4])
                    r.start()
                    rdmas.append(r)
                for dr in DIRS:
                    bb = (pos - dr["sgn"] * (1 + h)) % N_DEV
                    loads.append(start_load(dr, bb))
                for r in rdmas:
                    r.wait()
                for cp in loads:
                    cp.wait()
                for dr in DIRS:
                    d = dr["d"]
                    psum = rs_recv[d, g % 4, :, :].astype(f32) + ld[d, :, :]
                    if h < N_DEV - 2:
                        rs_send[d, (h + 1) % 2, :, :] = psum.astype(bf16)
                    else:
                        r_ = jnp.maximum(psum, 0.0)
                        slot = acc_write(
                            dr, jnp.tanh(psum) * psum * psum + r_ * r_ * r_)
                        start_store(dr, slot, (pos + dr["sgn"]) % N_DEV)
                        ag_bf[d, :, :] = acc[d, slot, :, :].astype(bf16)

            for h in range(N_DEV - 1):
                g = 3 * _c + h
                rdmas = []
                for dr in DIRS:
                    d = dr["d"]
                    src = ag_bf.at[d] if h == 0 else ag_recv.at[d, (g - 1) % 4]
                    r = rdma(dr, src, ag_recv.at[d, g % 4],
                             ag_ssem, h % 2, ag_rsem.at[d, g % 4])
                    r.start()
                    rdmas.append(r)
                for r in rdmas:
                    r.wait()
                for dr in DIRS:
                    d = dr["d"]
                    bb = (pos - dr["sgn"] * h) % N_DEV
                    slot = acc_write(dr, ag_recv[d, g % 4, :, :].astype(f32))
                    start_store(dr, slot, bb)

        for cp in pending_st.values():
            cp.wait()

    return pl.pallas_call(
        body,
        out_shape=jax.ShapeDtypeStruct((M, N), f32),
        in_specs=[pl.BlockSpec(memory_space=pl.ANY)],
        out_specs=pl.BlockSpec(memory_space=pl.ANY),
        scratch_shapes=[
            pltpu.VMEM((2, C, HN), f32),
            pltpu.VMEM((2, 2, C, HN), bf16),
            pltpu.VMEM((2, 4, C, HN), bf16),
            pltpu.VMEM((2, C, HN), bf16),
            pltpu.VMEM((2, 4, C, HN), bf16),
            pltpu.VMEM((2, 2, C, HN), f32),
            pltpu.SemaphoreType.DMA((2, 2)),
            pltpu.SemaphoreType.DMA((2, 4)),
            pltpu.SemaphoreType.DMA((2, 2)),
            pltpu.SemaphoreType.DMA((2, 4)),
            pltpu.SemaphoreType.DMA((2,)),
            pltpu.SemaphoreType.DMA((2, 2)),
        ],
        compiler_params=pltpu.CompilerParams(
            collective_id=0, vmem_limit_bytes=64 * 1024 * 1024),
    )(t)


# device time: 300971 ns/iter; 2.3413x vs baseline; 1.3158x over previous
import jax
import jax.numpy as jnp
from jax import lax
from jax.experimental import pallas as pl
from jax.experimental.pallas import tpu as pltpu

N_DEV = 4
M = 8192
N = 2048
HN = N // 2
MB = M // N_DEV
C = 512
N_SUB = MB // C
N_HOP = N_DEV - 1

f32 = jnp.float32
bf16 = jnp.bfloat16


def kernel(t):
    def body(t_ref, out_ref, ld, rs_send, rs_recv, ag_bf, ag_recv,
             rs_ssem, rs_rsem, ag_ssem, ag_rsem, cr_rs, cr_ag,
             ld_sem, st_sem):
        pos = lax.axis_index("i")
        left = (pos - 1) % N_DEV
        right = (pos + 1) % N_DEV

        barrier_sem = pltpu.get_barrier_semaphore()
        for nbr in (left, right):
            pl.semaphore_signal(
                barrier_sem, inc=1,
                device_id=(nbr,), device_id_type=pl.DeviceIdType.MESH,
            )
        pl.semaphore_wait(barrier_sem, 2)

        DIRS = (
            dict(d=0, col0=0, sgn=1, down=right, up=left),
            dict(d=1, col0=HN, sgn=-1, down=left, up=right),
        )

        def sched(dr):
            s = [(pos, c) for c in range(N_SUB)]
            for h in range(N_HOP):
                for c in range(N_SUB):
                    s.append(((pos - dr["sgn"] * (1 + h)) % N_DEV, c))
            return s

        SCHED = {dr["d"]: sched(dr) for dr in DIRS}
        ld_desc = {0: [], 1: []}

        def start_next_load(dr):
            d = dr["d"]
            j = len(ld_desc[d])
            if j < len(SCHED[d]):
                b, c = SCHED[d][j]
                cp = pltpu.make_async_copy(
                    t_ref.at[pl.ds(b * MB + c * C, C),
                             pl.ds(dr["col0"], HN)],
                    ld.at[d, j % 2], ld_sem.at[d, j % 2])
                cp.start()
                ld_desc[d].append(cp)

        def slot_of(h, c):
            return (N_SUB * h + c) % 8

        rs_msg, ag_msg = {}, {}

        def issue_rs(dr, h, c):
            d = dr["d"]
            desc = pltpu.make_async_remote_copy(
                src_ref=rs_send.at[d, c],
                dst_ref=rs_recv.at[d, slot_of(h, c)],
                send_sem=rs_ssem.at[d, c],
                recv_sem=rs_rsem.at[d, slot_of(h, c)],
                device_id=(dr["down"],),
                device_id_type=pl.DeviceIdType.MESH)
            desc.start()
            rs_msg[(d, h, c)] = desc

        def issue_ag(dr, h, c):
            d = dr["d"]
            src = ag_bf.at[d, c] if h == 0 else ag_recv.at[d, slot_of(h - 1, c)]
            desc = pltpu.make_async_remote_copy(
                src_ref=src,
                dst_ref=ag_recv.at[d, slot_of(h, c)],
                send_sem=ag_ssem.at[d, c],
                recv_sem=ag_rsem.at[d, slot_of(h, c)],
                device_id=(dr["down"],),
                device_id_type=pl.DeviceIdType.MESH)
            desc.start()
            ag_msg[(d, h, c)] = desc

        pend_st = {}

        def issue_store(dr, src, b, c):
            d = dr["d"]
            if (d, c) in pend_st:
                pend_st.pop((d, c)).wait()
            cp = pltpu.make_async_copy(
                src,
                out_ref.at[pl.ds(b * MB + c * C, C), pl.ds(dr["col0"], HN)],
                st_sem.at[d, c])
            cp.start()
            pend_st[(d, c)] = cp

        for dr in DIRS:
            start_next_load(dr)
            start_next_load(dr)
        for c in range(N_SUB):
            for dr in DIRS:
                d = dr["d"]
                ld_desc[d][c].wait()
                rs_send[d, c, :, :] = ld[d, c % 2, :, :].astype(bf16)
                start_next_load(dr)
            for dr in DIRS:
                issue_rs(dr, 0, c)

        for h in range(N_HOP):
            for c in range(N_SUB):
                for dr in DIRS:
                    rs_msg[(dr["d"], h, c)].wait_recv()
                for dr in DIRS:
                    d = dr["d"]
                    j = N_SUB + N_SUB * h + c
                    ld_desc[d][j].wait()
                    psum = (rs_recv[d, slot_of(h, c), :, :].astype(f32)
                            + ld[d, j % 2, :, :])
                    start_next_load(dr)
                    if h == 0:
                        pl.semaphore_signal(
                            cr_rs.at[d, c], inc=1,
                            device_id=(dr["up"],),
                            device_id_type=pl.DeviceIdType.MESH)
                    if h < N_HOP - 1:
                        rs_msg[(d, h, c)].wait_send()
                        rs_send[d, c, :, :] = psum.astype(bf16)
                        if h + 1 == N_HOP - 1:
                            pl.semaphore_wait(cr_rs.at[d, c], 1)
                        issue_rs(dr, h + 1, c)
                    else:
                        r_ = jnp.maximum(psum, 0.0)
                        ag_bf[d, c, :, :] = (
                            jnp.tanh(psum) * psum * psum + r_ * r_ * r_
                        ).astype(bf16)
                        issue_ag(dr, 0, c)
                        issue_store(dr, ag_bf.at[d, c],
                                    (pos + dr["sgn"]) % N_DEV, c)

        for h in range(N_HOP):
            for c in range(N_SUB):
                for dr in DIRS:
                    ag_msg[(dr["d"], h, c)].wait_recv()
                for dr in DIRS:
                    d = dr["d"]
                    bb = (pos - dr["sgn"] * h) % N_DEV
                    if h == 0:
                        ag_msg[(d, 0, c)].wait_send()
                        issue_store(dr, ag_recv.at[d, slot_of(0, c)], bb, c)
                        issue_ag(dr, 1, c)
                    elif h == 1:
                        ag_msg[(d, 1, c)].wait_send()
                        issue_store(dr, ag_recv.at[d, slot_of(1, c)], bb, c)
                        pl.semaphore_signal(
                            cr_ag.at[d, c], inc=1,
                            device_id=(dr["up"],),
                            device_id_type=pl.DeviceIdType.MESH)
                        pl.semaphore_wait(cr_ag.at[d, c], 1)
                        issue_ag(dr, 2, c)
                    else:
                        issue_store(dr, ag_recv.at[d, slot_of(2, c)], bb, c)

        for c in range(N_SUB):
            for dr in DIRS:
                d = dr["d"]
                rs_msg[(d, N_HOP - 1, c)].wait_send()
                ag_msg[(d, N_HOP - 1, c)].wait_send()
        for cp in pend_st.values():
            cp.wait()

    return pl.pallas_call(
        body,
        out_shape=jax.ShapeDtypeStruct((M, N), bf16),
        in_specs=[pl.BlockSpec(memory_space=pl.ANY)],
        out_specs=pl.BlockSpec(memory_space=pl.ANY),
        scratch_shapes=[
            pltpu.VMEM((2, 2, C, HN), f32),
            pltpu.VMEM((2, N_SUB, C, HN), bf16),
            pltpu.VMEM((2, 8, C, HN), bf16),
            pltpu.VMEM((2, N_SUB, C, HN), bf16),
            pltpu.VMEM((2, 8, C, HN), bf16),
            pltpu.SemaphoreType.DMA((2, N_SUB)),
            pltpu.SemaphoreType.DMA((2, 8)),
            pltpu.SemaphoreType.DMA((2, N_SUB)),
            pltpu.SemaphoreType.DMA((2, 8)),
            pltpu.SemaphoreType.REGULAR((2, N_SUB)),
            pltpu.SemaphoreType.REGULAR((2, N_SUB)),
            pltpu.SemaphoreType.DMA((2, 2)),
            pltpu.SemaphoreType.DMA((2, N_SUB)),
        ],
        compiler_params=pltpu.CompilerParams(
            collective_id=0, vmem_limit_bytes=96 * 1024 * 1024),
    )(t)


# device time: 300285 ns/iter; 2.3467x vs baseline; 1.0023x over previous
import jax
import jax.numpy as jnp
from jax import lax
from jax.experimental import pallas as pl
from jax.experimental.pallas import tpu as pltpu

N_DEV = 4
M = 8192
N = 2048
HN = N // 2
MB = M // N_DEV
C = 256
N_SUB = MB // C
N_HOP = N_DEV - 1
SLOTS = 2 * N_SUB

f32 = jnp.float32
bf16 = jnp.bfloat16


def kernel(t):
    def body(t_ref, out_ref, ld, rs_send, rs_recv, ag_bf, ag_recv,
             rs_ssem, rs_rsem, ag_ssem, ag_rsem, cr_rs, cr_ag,
             ld_sem, st_sem):
        pos = lax.axis_index("i")
        left = (pos - 1) % N_DEV
        right = (pos + 1) % N_DEV

        barrier_sem = pltpu.get_barrier_semaphore()
        for nbr in (left, right):
            pl.semaphore_signal(
                barrier_sem, inc=1,
                device_id=(nbr,), device_id_type=pl.DeviceIdType.MESH,
            )
        pl.semaphore_wait(barrier_sem, 2)

        DIRS = (
            dict(d=0, col0=0, sgn=1, down=right, up=left),
            dict(d=1, col0=HN, sgn=-1, down=left, up=right),
        )

        def sched(dr):
            s = [(pos, c) for c in range(N_SUB)]
            for h in range(N_HOP):
                for c in range(N_SUB):
                    s.append(((pos - dr["sgn"] * (1 + h)) % N_DEV, c))
            return s

        SCHED = {dr["d"]: sched(dr) for dr in DIRS}
        ld_desc = {0: [], 1: []}

        def start_next_load(dr):
            d = dr["d"]
            j = len(ld_desc[d])
            if j < len(SCHED[d]):
                b, c = SCHED[d][j]
                cp = pltpu.make_async_copy(
                    t_ref.at[pl.ds(b * MB + c * C, C),
                             pl.ds(dr["col0"], HN)],
                    ld.at[d, j % 2], ld_sem.at[d, j % 2])
                cp.start()
                ld_desc[d].append(cp)

        def slot_of(h, c):
            return (N_SUB * h + c) % SLOTS

        rs_msg, ag_msg = {}, {}

        def issue_rs(dr, h, c):
            d = dr["d"]
            desc = pltpu.make_async_remote_copy(
                src_ref=rs_send.at[d, c],
                dst_ref=rs_recv.at[d, slot_of(h, c)],
                send_sem=rs_ssem.at[d, c],
                recv_sem=rs_rsem.at[d, slot_of(h, c)],
                device_id=(dr["down"],),
                device_id_type=pl.DeviceIdType.MESH)
            desc.start()
            rs_msg[(d, h, c)] = desc

        def issue_ag(dr, h, c):
            d = dr["d"]
            src = ag_bf.at[d, c] if h == 0 else ag_recv.at[d, slot_of(h - 1, c)]
            desc = pltpu.make_async_remote_copy(
                src_ref=src,
                dst_ref=ag_recv.at[d, slot_of(h, c)],
                send_sem=ag_ssem.at[d, c],
                recv_sem=ag_rsem.at[d, slot_of(h, c)],
                device_id=(dr["down"],),
                device_id_type=pl.DeviceIdType.MESH)
            desc.start()
            ag_msg[(d, h, c)] = desc

        pend_st = {}

        def issue_store(dr, src, b, c):
            d = dr["d"]
            if (d, c) in pend_st:
                pend_st.pop((d, c)).wait()
            cp = pltpu.make_async_copy(
                src,
                out_ref.at[pl.ds(b * MB + c * C, C), pl.ds(dr["col0"], HN)],
                st_sem.at[d, c])
            cp.start()
            pend_st[(d, c)] = cp

        for dr in DIRS:
            start_next_load(dr)
            start_next_load(dr)
        for c in range(N_SUB):
            for dr in DIRS:
                d = dr["d"]
                ld_desc[d][c].wait()
                rs_send[d, c, :, :] = ld[d, c % 2, :, :].astype(bf16)
                start_next_load(dr)
            for dr in DIRS:
                issue_rs(dr, 0, c)

        for h in range(N_HOP):
            for c in range(N_SUB):
                for dr in DIRS:
                    rs_msg[(dr["d"], h, c)].wait_recv()
                for dr in DIRS:
                    d = dr["d"]
                    j = N_SUB + N_SUB * h + c
                    ld_desc[d][j].wait()
                    psum = (rs_recv[d, slot_of(h, c), :, :].astype(f32)
                            + ld[d, j % 2, :, :])
                    start_next_load(dr)
                    if h == 0:
                        pl.semaphore_signal(
                            cr_rs.at[d, c], inc=1,
                            device_id=(dr["up"],),
                            device_id_type=pl.DeviceIdType.MESH)
                    if h < N_HOP - 1:
                        rs_msg[(d, h, c)].wait_send()
                        rs_send[d, c, :, :] = psum.astype(bf16)
                        if h + 1 == N_HOP - 1:
                            pl.semaphore_wait(cr_rs.at[d, c], 1)
                        issue_rs(dr, h + 1, c)
                    else:
                        r_ = jnp.maximum(psum, 0.0)
                        ag_bf[d, c, :, :] = (
                            jnp.tanh(psum) * psum * psum + r_ * r_ * r_
                        ).astype(bf16)
                        issue_ag(dr, 0, c)
                        issue_store(dr, ag_bf.at[d, c],
                                    (pos + dr["sgn"]) % N_DEV, c)

        for h in range(N_HOP):
            for c in range(N_SUB):
                for dr in DIRS:
                    ag_msg[(dr["d"], h, c)].wait_recv()
                for dr in DIRS:
                    d = dr["d"]
                    bb = (pos - dr["sgn"] * h) % N_DEV
                    if h == 0:
                        ag_msg[(d, 0, c)].wait_send()
                        issue_store(dr, ag_recv.at[d, slot_of(0, c)], bb, c)
                        issue_ag(dr, 1, c)
                    elif h == 1:
                        ag_msg[(d, 1, c)].wait_send()
                        issue_store(dr, ag_recv.at[d, slot_of(1, c)], bb, c)
                        pl.semaphore_signal(
                            cr_ag.at[d, c], inc=1,
                            device_id=(dr["up"],),
                            device_id_type=pl.DeviceIdType.MESH)
                        pl.semaphore_wait(cr_ag.at[d, c], 1)
                        issue_ag(dr, 2, c)
                    else:
                        issue_store(dr, ag_recv.at[d, slot_of(2, c)], bb, c)

        for c in range(N_SUB):
            for dr in DIRS:
                d = dr["d"]
                rs_msg[(d, N_HOP - 1, c)].wait_send()
                ag_msg[(d, N_HOP - 1, c)].wait_send()
        for cp in pend_st.values():
            cp.wait()

    return pl.pallas_call(
        body,
        out_shape=jax.ShapeDtypeStruct((M, N), bf16),
        in_specs=[pl.BlockSpec(memory_space=pl.ANY)],
        out_specs=pl.BlockSpec(memory_space=pl.ANY),
        scratch_shapes=[
            pltpu.VMEM((2, 2, C, HN), f32),
            pltpu.VMEM((2, N_SUB, C, HN), bf16),
            pltpu.VMEM((2, SLOTS, C, HN), bf16),
            pltpu.VMEM((2, N_SUB, C, HN), bf16),
            pltpu.VMEM((2, SLOTS, C, HN), bf16),
            pltpu.SemaphoreType.DMA((2, N_SUB)),
            pltpu.SemaphoreType.DMA((2, SLOTS)),
            pltpu.SemaphoreType.DMA((2, N_SUB)),
            pltpu.SemaphoreType.DMA((2, SLOTS)),
            pltpu.SemaphoreType.REGULAR((2, N_SUB)),
            pltpu.SemaphoreType.REGULAR((2, N_SUB)),
            pltpu.SemaphoreType.DMA((2, 2)),
            pltpu.SemaphoreType.DMA((2, N_SUB)),
        ],
        compiler_params=pltpu.CompilerParams(
            collective_id=0, vmem_limit_bytes=96 * 1024 * 1024),
    )(t)


# device time: 299203 ns/iter; 2.3552x vs baseline; 1.0036x over previous
import jax
import jax.numpy as jnp
from jax import lax
from jax.experimental import pallas as pl
from jax.experimental.pallas import tpu as pltpu

N_DEV = 4
M = 8192
N = 2048
HN = N // 2
MB = M // N_DEV
C = 256
N_SUB = MB // C
N_HOP = N_DEV - 1
SLOTS = 2 * N_SUB

f32 = jnp.float32
bf16 = jnp.bfloat16


def kernel(t):
    def body(t_ref, out_ref, ld, rs_send, rs_recv, ag_bf, ag_recv,
             rs_ssem, rs_rsem, ag_ssem, ag_rsem, cr_rs, cr_ag,
             ld_sem, st_sem):
        pos = lax.axis_index("i")
        left = (pos - 1) % N_DEV
        right = (pos + 1) % N_DEV

        DIRS = (
            dict(d=0, col0=0, sgn=1, down=right, up=left),
            dict(d=1, col0=HN, sgn=-1, down=left, up=right),
        )

        def sched(dr):
            s = [(pos, c) for c in range(N_SUB)]
            for h in range(N_HOP):
                for c in range(N_SUB):
                    s.append(((pos - dr["sgn"] * (1 + h)) % N_DEV, c))
            return s

        SCHED = {dr["d"]: sched(dr) for dr in DIRS}
        ld_desc = {0: [], 1: []}

        def start_next_load(dr):
            d = dr["d"]
            j = len(ld_desc[d])
            if j < len(SCHED[d]):
                b, c = SCHED[d][j]
                cp = pltpu.make_async_copy(
                    t_ref.at[pl.ds(b * MB + c * C, C),
                             pl.ds(dr["col0"], HN)],
                    ld.at[d, j % 2], ld_sem.at[d, j % 2])
                cp.start()
                ld_desc[d].append(cp)

        def slot_of(h, c):
            return (N_SUB * h + c) % SLOTS

        rs_msg, ag_msg = {}, {}

        def issue_rs(dr, h, c):
            d = dr["d"]
            desc = pltpu.make_async_remote_copy(
                src_ref=rs_send.at[d, c],
                dst_ref=rs_recv.at[d, slot_of(h, c)],
                send_sem=rs_ssem.at[d, c],
                recv_sem=rs_rsem.at[d, slot_of(h, c)],
                device_id=(dr["down"],),
                device_id_type=pl.DeviceIdType.MESH)
            desc.start()
            rs_msg[(d, h, c)] = desc

        def issue_ag(dr, h, c):
            d = dr["d"]
            src = ag_bf.at[d, c] if h == 0 else ag_recv.at[d, slot_of(h - 1, c)]
            desc = pltpu.make_async_remote_copy(
                src_ref=src,
                dst_ref=ag_recv.at[d, slot_of(h, c)],
                send_sem=ag_ssem.at[d, c],
                recv_sem=ag_rsem.at[d, slot_of(h, c)],
                device_id=(dr["down"],),
                device_id_type=pl.DeviceIdType.MESH)
            desc.start()
            ag_msg[(d, h, c)] = desc

        pend_st = {}

        def issue_store(dr, src, b, c):
            d = dr["d"]
            if (d, c) in pend_st:
                pend_st.pop((d, c)).wait()
            cp = pltpu.make_async_copy(
                src,
                out_ref.at[pl.ds(b * MB + c * C, C), pl.ds(dr["col0"], HN)],
                st_sem.at[d, c])
            cp.start()
            pend_st[(d, c)] = cp

        for dr in DIRS:
            start_next_load(dr)
            start_next_load(dr)

        barrier_sem = pltpu.get_barrier_semaphore()
        for nbr in (left, right):
            pl.semaphore_signal(
                barrier_sem, inc=1,
                device_id=(nbr,), device_id_type=pl.DeviceIdType.MESH,
            )
        pl.semaphore_wait(barrier_sem, 2)

        for c in range(N_SUB):
            for dr in DIRS:
                d = dr["d"]
                ld_desc[d][c].wait()
                rs_send[d, c, :, :] = ld[d, c % 2, :, :].astype(bf16)
                start_next_load(dr)
            for dr in DIRS:
                issue_rs(dr, 0, c)

        for h in range(N_HOP):
            for c in range(N_SUB):
                for dr in DIRS:
                    rs_msg[(dr["d"], h, c)].wait_recv()
                for dr in DIRS:
                    d = dr["d"]
                    j = N_SUB + N_SUB * h + c
                    ld_desc[d][j].wait()
                    psum = (rs_recv[d, slot_of(h, c), :, :].astype(f32)
                            + ld[d, j % 2, :, :])
                    start_next_load(dr)
                    if h == 0:
                        pl.semaphore_signal(
                            cr_rs.at[d, c], inc=1,
                            device_id=(dr["up"],),
                            device_id_type=pl.DeviceIdType.MESH)
                    if h < N_HOP - 1:
                        rs_msg[(d, h, c)].wait_send()
                        rs_send[d, c, :, :] = psum.astype(bf16)
                        if h + 1 == N_HOP - 1:
                            pl.semaphore_wait(cr_rs.at[d, c], 1)
                        issue_rs(dr, h + 1, c)
                    else:
                        r_ = jnp.maximum(psum, 0.0)
                        ag_bf[d, c, :, :] = (
                            jnp.tanh(psum) * psum * psum + r_ * r_ * r_
                        ).astype(bf16)
                        issue_ag(dr, 0, c)
                        issue_store(dr, ag_bf.at[d, c],
                                    (pos + dr["sgn"]) % N_DEV, c)

        for h in range(N_HOP):
            for c in range(N_SUB):
                for dr in DIRS:
                    ag_msg[(dr["d"], h, c)].wait_recv()
                for dr in DIRS:
                    d = dr["d"]
                    bb = (pos - dr["sgn"] * h) % N_DEV
                    if h == 0:
                        ag_msg[(d, 0, c)].wait_send()
                        issue_store(dr, ag_recv.at[d, slot_of(0, c)], bb, c)
                        issue_ag(dr, 1, c)
                    elif h == 1:
                        ag_msg[(d, 1, c)].wait_send()
                        issue_store(dr, ag_recv.at[d, slot_of(1, c)], bb, c)
                        pl.semaphore_signal(
                            cr_ag.at[d, c], inc=1,
                            device_id=(dr["up"],),
                            device_id_type=pl.DeviceIdType.MESH)
                        pl.semaphore_wait(cr_ag.at[d, c], 1)
                        issue_ag(dr, 2, c)
                    else:
                        issue_store(dr, ag_recv.at[d, slot_of(2, c)], bb, c)

        for c in range(N_SUB):
            for dr in DIRS:
                d = dr["d"]
                rs_msg[(d, N_HOP - 1, c)].wait_send()
                ag_msg[(d, N_HOP - 1, c)].wait_send()
        for cp in pend_st.values():
            cp.wait()

    return pl.pallas_call(
        body,
        out_shape=jax.ShapeDtypeStruct((M, N), bf16),
        in_specs=[pl.BlockSpec(memory_space=pl.ANY)],
        out_specs=pl.BlockSpec(memory_space=pl.ANY),
        scratch_shapes=[
            pltpu.VMEM((2, 2, C, HN), f32),
            pltpu.VMEM((2, N_SUB, C, HN), bf16),
            pltpu.VMEM((2, SLOTS, C, HN), bf16),
            pltpu.VMEM((2, N_SUB, C, HN), bf16),
            pltpu.VMEM((2, SLOTS, C, HN), bf16),
            pltpu.SemaphoreType.DMA((2, N_SUB)),
            pltpu.SemaphoreType.DMA((2, SLOTS)),
            pltpu.SemaphoreType.DMA((2, N_SUB)),
            pltpu.SemaphoreType.DMA((2, SLOTS)),
            pltpu.SemaphoreType.REGULAR((2, N_SUB)),
            pltpu.SemaphoreType.REGULAR((2, N_SUB)),
            pltpu.SemaphoreType.DMA((2, 2)),
            pltpu.SemaphoreType.DMA((2, N_SUB)),
        ],
        compiler_params=pltpu.CompilerParams(
            collective_id=0, vmem_limit_bytes=96 * 1024 * 1024),
    )(t)
